# Initial kernel scaffold; baseline (speedup 1.0000x reference)
import jax
import jax.numpy as jnp
from jax.experimental import pallas as pl


def kernel(h, e, edge_index, W1, b1, W2, b2, W3, b3, bias, gamma, beta):
    raise NotImplementedError("write your pallas kernel here")


# speedup vs baseline: 3.4836x; 3.4836x over previous
import functools

import jax
import jax.numpy as jnp
from jax import lax
from jax.experimental import pallas as pl
from jax.experimental.pallas import tpu as pltpu
from jax.experimental.pallas import tpu_sc as plsc

N = 10000
E = 160000
D = 16
H = 128
DD = D * D

NC = 2
NS = 16
NW = NC * NS
EPW = E // NW
CH = 125
NCH = EPW // CH
NP = 10240
RPS = NP // NS

EB = 2000
_MESH = plsc.VectorSubcoreMesh(core_axis_name="c", subcore_axis_name="s")


def _zero_rows(ref, n):
    def body(i, carry):
        ref[i, :] = jnp.zeros((D,), jnp.float32)
        return carry
    lax.fori_loop(0, n, body, 0)


@functools.partial(
    pl.kernel,
    out_type=(
        jax.ShapeDtypeStruct((NW * NCH, CH, D), jnp.float32),
        jax.ShapeDtypeStruct((NC, NP, D), jnp.float32),
    ),
    scratch_types=[
        pltpu.VMEM((NCH, CH), jnp.int32),
        pltpu.VMEM((NCH, CH), jnp.int32),
        pltpu.VMEM((NCH, CH, D), jnp.float32),
        pltpu.VMEM((CH, D), jnp.float32),
        pltpu.VMEM((RPS, D), jnp.float32),
        pltpu.SemaphoreType.DMA,
        pltpu.VMEM_SHARED((NP, D), jnp.float32),
    ],
    mesh=_MESH,
    compiler_params=pltpu.CompilerParams(use_tc_tiling_on_sc=False),
)
def _sc_gather_counts(x_hbm, src_hbm, dst_hbm, xs_out, cnt_out,
                      sidx, didx, rows, obuf, zbuf, sem, cnt_sh):
    c = lax.axis_index("c")
    s = lax.axis_index("s")
    wid = s * NC + c
    pltpu.sync_copy(src_hbm.at[pl.ds(wid * NCH, NCH)], sidx)
    pltpu.sync_copy(dst_hbm.at[pl.ds(wid * NCH, NCH)], didx)

    def initb(i, carry):
        obuf[i, :] = jnp.ones((D,), jnp.float32)
        return carry
    lax.fori_loop(0, CH, initb, 0)
    _zero_rows(zbuf, RPS)
    pltpu.sync_copy(zbuf, cnt_sh.at[pl.ds(s * RPS, RPS)])
    plsc.subcore_barrier()

    def chunk(j, carry):
        pltpu.async_copy(x_hbm.at[sidx.at[j]], rows.at[j], sem).wait()
        pltpu.sync_copy(obuf, cnt_sh.at[didx.at[j]], add=True)
        return carry
    lax.fori_loop(0, NCH, chunk, 0)

    pltpu.sync_copy(rows, xs_out.at[pl.ds(wid * NCH, NCH)])
    plsc.subcore_barrier()
    pltpu.sync_copy(cnt_sh.at[pl.ds(s * RPS, RPS)],
                    cnt_out.at[c].at[pl.ds(s * RPS, RPS)])


@functools.partial(
    pl.kernel,
    out_type=jax.ShapeDtypeStruct((NW * NCH, CH, D), jnp.float32),
    scratch_types=[
        pltpu.VMEM((NCH, CH), jnp.int32),
        pltpu.VMEM((NCH, CH, D), jnp.float32),
        pltpu.SemaphoreType.DMA,
    ],
    mesh=_MESH,
    compiler_params=pltpu.CompilerParams(use_tc_tiling_on_sc=False),
)
def _sc_gather(x_hbm, src_hbm, xs_out, sidx, rows, sem):
    c = lax.axis_index("c")
    s = lax.axis_index("s")
    wid = s * NC + c
    pltpu.sync_copy(src_hbm.at[pl.ds(wid * NCH, NCH)], sidx)

    def chunk(j, carry):
        pltpu.async_copy(x_hbm.at[sidx.at[j]], rows.at[j], sem).wait()
        return carry
    lax.fori_loop(0, NCH, chunk, 0)

    pltpu.sync_copy(rows, xs_out.at[pl.ds(wid * NCH, NCH)])


@functools.partial(
    pl.kernel,
    out_type=jax.ShapeDtypeStruct((NC, NP, D), jnp.float32),
    scratch_types=[
        pltpu.VMEM((NCH, CH), jnp.int32),
        pltpu.VMEM((NCH, CH, D), jnp.float32),
        pltpu.VMEM((RPS, D), jnp.float32),
        pltpu.VMEM_SHARED((NP, D), jnp.float32),
    ],
    mesh=_MESH,
    compiler_params=pltpu.CompilerParams(use_tc_tiling_on_sc=False),
)
def _sc_scatter(dst_hbm, msg_hbm, agg_out, didx, buf, zbuf, agg_sh):
    c = lax.axis_index("c")
    s = lax.axis_index("s")
    wid = s * NC + c
    pltpu.sync_copy(dst_hbm.at[pl.ds(wid * NCH, NCH)], didx)
    pltpu.sync_copy(msg_hbm.at[pl.ds(wid * NCH, NCH)], buf)
    _zero_rows(zbuf, RPS)
    pltpu.sync_copy(zbuf, agg_sh.at[pl.ds(s * RPS, RPS)])
    plsc.subcore_barrier()

    def chunk(j, carry):
        pltpu.sync_copy(buf.at[j], agg_sh.at[didx.at[j]], add=True)
        return carry
    lax.fori_loop(0, NCH, chunk, 0)

    plsc.subcore_barrier()
    pltpu.sync_copy(agg_sh.at[pl.ds(s * RPS, RPS)],
                    agg_out.at[c].at[pl.ds(s * RPS, RPS)])


def _bn_affine(x, gamma, beta):
    mu = jnp.mean(x, axis=0, keepdims=True)
    var = jnp.mean((x - mu) ** 2, axis=0, keepdims=True)
    a = gamma / jnp.sqrt(var + 1e-5)
    return jnp.concatenate([a, beta - mu * a], axis=0)


def _stats_body(h_ref, gamma_ref, beta_ref, ac_ref):
    ac_ref[...] = _bn_affine(h_ref[...], gamma_ref[...], beta_ref[...])


def _stats(h, gamma2, beta2):
    return pl.pallas_call(
        _stats_body,
        out_shape=jax.ShapeDtypeStruct((2, D), jnp.float32),
    )(h, gamma2, beta2)


def _dense_body(e_ref, xs_ref, w1_ref, b1_ref, w2_ref, b2_ref, w3_ref, b3_ref,
                ac_ref, r_ref, s_ref, msg_ref):
    x = xs_ref[...] * ac_ref[0:1, :] + ac_ref[1:2, :]
    h1 = jnp.maximum(e_ref[...] * w1_ref[...] + b1_ref[...], 0.0)
    h2 = jnp.maximum(
        jnp.dot(h1, w2_ref[...], preferred_element_type=jnp.float32)
        + b2_ref[...], 0.0)
    w = (jnp.dot(h2, w3_ref[...], preferred_element_type=jnp.float32)
         + b3_ref[...])
    xr = jnp.dot(x, r_ref[...], preferred_element_type=jnp.float32)
    msg_ref[...] = jnp.dot(xr * w, s_ref[...],
                           preferred_element_type=jnp.float32)


def _dense(e, xs, w1, b1r, w2, b2r, w3, b3r, ac, r, s):
    full = lambda shape: pl.BlockSpec(shape, lambda i: (0, 0))
    return pl.pallas_call(
        _dense_body,
        grid=(E // EB,),
        in_specs=[
            pl.BlockSpec((EB, 1), lambda i: (i, 0)),
            pl.BlockSpec((EB, D), lambda i: (i, 0)),
            full((1, H)), full((1, H)),
            full((H, H)), full((1, H)),
            full((H, DD)), full((1, DD)),
            full((2, D)), full((D, DD)), full((DD, D)),
        ],
        out_specs=pl.BlockSpec((EB, D), lambda i: (i, 0)),
        out_shape=jax.ShapeDtypeStruct((E, D), jnp.float32),
    )(e, xs, w1, b1r, w2, b2r, w3, b3r, ac, r, s)


def _update_body(aggp_ref, cntp_ref, bias_ref, hin_ref, gamma_ref, beta_ref,
                 hout_ref, ac_ref):
    agg = aggp_ref[0, :N, :] + aggp_ref[1, :N, :]
    cnt = cntp_ref[0, :N, 0:1] + cntp_ref[1, :N, 0:1]
    denom = jnp.maximum(cnt, 1.0)
    hnew = agg / denom + bias_ref[...] + hin_ref[...]
    hout_ref[...] = hnew
    ac_ref[...] = _bn_affine(hnew, gamma_ref[...], beta_ref[...])


def _update(aggp, cntp, bias2, hin, gamma2, beta2):
    return pl.pallas_call(
        _update_body,
        out_shape=(
            jax.ShapeDtypeStruct((N, D), jnp.float32),
            jax.ShapeDtypeStruct((2, D), jnp.float32),
        ),
    )(aggp, cntp, bias2, hin, gamma2, beta2)


def _final_body(aggp_ref, cntp_ref, bias_ref, hin_ref, hout_ref):
    agg = aggp_ref[0, :N, :] + aggp_ref[1, :N, :]
    cnt = cntp_ref[0, :N, 0:1] + cntp_ref[1, :N, 0:1]
    denom = jnp.maximum(cnt, 1.0)
    hout_ref[...] = agg / denom + bias_ref[...] + hin_ref[...]


def _final(aggp, cntp, bias2, hin):
    return pl.pallas_call(
        _final_body,
        out_shape=jax.ShapeDtypeStruct((N, D), jnp.float32),
    )(aggp, cntp, bias2, hin)


def kernel(h, e, edge_index, W1, b1, W2, b2, W3, b3, bias, gamma, beta):
    src2 = edge_index[1].reshape(NW * NCH, CH)
    dst2 = edge_index[0].reshape(NW * NCH, CH)
    b1r = b1.reshape(1, H)
    b2r = b2.reshape(1, H)
    b3r = b3.reshape(1, DD)
    bias2 = bias.reshape(1, D)
    gamma2 = gamma.reshape(1, D)
    beta2 = beta.reshape(1, D)
    r = jnp.kron(jnp.eye(D, dtype=jnp.float32),
                 jnp.ones((1, D), jnp.float32))
    s = jnp.kron(jnp.ones((D, 1), jnp.float32),
                 jnp.eye(D, dtype=jnp.float32))

    ac1 = _stats(h, gamma2, beta2)
    xs1, cntp = _sc_gather_counts(h, src2, dst2)
    msg1 = _dense(e, xs1.reshape(E, D), W1, b1r, W2, b2r, W3, b3r, ac1, r, s)
    aggp1 = _sc_scatter(dst2, msg1.reshape(NW * NCH, CH, D))
    h2, ac2 = _update(aggp1, cntp, bias2, h, gamma2, beta2)
    xs2 = _sc_gather(h2, src2)
    msg2 = _dense(e, xs2.reshape(E, D), W1, b1r, W2, b2r, W3, b3r, ac2, r, s)
    aggp2 = _sc_scatter(dst2, msg2.reshape(NW * NCH, CH, D))
    return _final(aggp2, cntp, bias2, h2)


# speedup vs baseline: 3.6283x; 1.0415x over previous
import functools

import jax
import jax.numpy as jnp
from jax import lax
from jax.experimental import pallas as pl
from jax.experimental.pallas import tpu as pltpu
from jax.experimental.pallas import tpu_sc as plsc

N = 10000
E = 160000
D = 16
H = 128
DD = D * D

NC = 2
NS = 16
NW = NC * NS
EPW = E // NW
CH = 125
NCH = EPW // CH
NP = 10240
RPS = NP // NS

EB = 2000
_MESH = plsc.VectorSubcoreMesh(core_axis_name="c", subcore_axis_name="s")
_SC_PARAMS = pltpu.CompilerParams(use_tc_tiling_on_sc=False)


def _fill_rows(ref, n, value):
    def body(i, carry):
        ref[i, :] = jnp.full((D,), value, jnp.float32)
        return carry
    lax.fori_loop(0, n, body, 0)


def _fire_drain(n, fire):
    def fire_body(j, carry):
        fire(j)
        return carry
    lax.fori_loop(0, n, fire_body, 0)

    def drain_body(j, carry):
        fire(0, wait=True)
        return carry
    lax.fori_loop(0, n, drain_body, 0)


@functools.partial(
    pl.kernel,
    out_type=(
        jax.ShapeDtypeStruct((NW * NCH, CH, D), jnp.float32),
        jax.ShapeDtypeStruct((NC, NP, D), jnp.float32),
    ),
    scratch_types=[
        pltpu.VMEM((NCH, CH), jnp.int32),
        pltpu.VMEM((NCH, CH), jnp.int32),
        pltpu.VMEM((NCH, CH, D), jnp.float32),
        pltpu.VMEM((CH, D), jnp.float32),
        pltpu.VMEM((RPS, D), jnp.float32),
        pltpu.SemaphoreType.DMA,
        pltpu.SemaphoreType.DMA,
        pltpu.VMEM_SHARED((NP, D), jnp.float32),
    ],
    mesh=_MESH,
    compiler_params=_SC_PARAMS,
)
def _sc_gather_counts(x_hbm, src_hbm, dst_hbm, xs_out, cnt_out,
                      sidx, didx, rows, obuf, zbuf, gsem, csem, cnt_sh):
    c = lax.axis_index("c")
    s = lax.axis_index("s")
    wid = s * NC + c
    pltpu.sync_copy(src_hbm.at[pl.ds(wid * NCH, NCH)], sidx)
    pltpu.sync_copy(dst_hbm.at[pl.ds(wid * NCH, NCH)], didx)
    _fill_rows(obuf, CH, 1.0)
    _fill_rows(zbuf, RPS, 0.0)
    pltpu.sync_copy(zbuf, cnt_sh.at[pl.ds(s * RPS, RPS)])
    plsc.subcore_barrier()

    def gath(j, wait=False):
        d = pltpu.make_async_copy(x_hbm.at[sidx.at[j]], rows.at[j], gsem)
        d.wait() if wait else d.start()
    _fire_drain(NCH, gath)

    def cadd(j, wait=False):
        d = pltpu.make_async_copy(obuf, cnt_sh.at[didx.at[j]], csem)
        d.wait() if wait else d.start(add=True)
    _fire_drain(NCH, cadd)

    pltpu.sync_copy(rows, xs_out.at[pl.ds(wid * NCH, NCH)])
    plsc.subcore_barrier()
    pltpu.sync_copy(cnt_sh.at[pl.ds(s * RPS, RPS)],
                    cnt_out.at[c].at[pl.ds(s * RPS, RPS)])


@functools.partial(
    pl.kernel,
    out_type=jax.ShapeDtypeStruct((NW * NCH, CH, D), jnp.float32),
    scratch_types=[
        pltpu.VMEM((NCH, CH), jnp.int32),
        pltpu.VMEM((NCH, CH, D), jnp.float32),
        pltpu.SemaphoreType.DMA,
    ],
    mesh=_MESH,
    compiler_params=_SC_PARAMS,
)
def _sc_gather(x_hbm, src_hbm, xs_out, sidx, rows, gsem):
    c = lax.axis_index("c")
    s = lax.axis_index("s")
    wid = s * NC + c
    pltpu.sync_copy(src_hbm.at[pl.ds(wid * NCH, NCH)], sidx)

    def gath(j, wait=False):
        d = pltpu.make_async_copy(x_hbm.at[sidx.at[j]], rows.at[j], gsem)
        d.wait() if wait else d.start()
    _fire_drain(NCH, gath)

    pltpu.sync_copy(rows, xs_out.at[pl.ds(wid * NCH, NCH)])


@functools.partial(
    pl.kernel,
    out_type=jax.ShapeDtypeStruct((NC, NP, D), jnp.float32),
    scratch_types=[
        pltpu.VMEM((NCH, CH), jnp.int32),
        pltpu.VMEM((NCH, CH, D), jnp.float32),
        pltpu.VMEM((RPS, D), jnp.float32),
        pltpu.SemaphoreType.DMA,
        pltpu.VMEM_SHARED((NP, D), jnp.float32),
    ],
    mesh=_MESH,
    compiler_params=_SC_PARAMS,
)
def _sc_scatter(dst_hbm, msg_hbm, agg_out, didx, buf, zbuf, asem, agg_sh):
    c = lax.axis_index("c")
    s = lax.axis_index("s")
    wid = s * NC + c
    pltpu.sync_copy(dst_hbm.at[pl.ds(wid * NCH, NCH)], didx)
    pltpu.sync_copy(msg_hbm.at[pl.ds(wid * NCH, NCH)], buf)
    _fill_rows(zbuf, RPS, 0.0)
    pltpu.sync_copy(zbuf, agg_sh.at[pl.ds(s * RPS, RPS)])
    plsc.subcore_barrier()

    def sadd(j, wait=False):
        d = pltpu.make_async_copy(buf.at[j], agg_sh.at[didx.at[j]], asem)
        d.wait() if wait else d.start(add=True)
    _fire_drain(NCH, sadd)

    plsc.subcore_barrier()
    pltpu.sync_copy(agg_sh.at[pl.ds(s * RPS, RPS)],
                    agg_out.at[c].at[pl.ds(s * RPS, RPS)])


def _bn_affine(x, gamma, beta):
    mu = jnp.mean(x, axis=0, keepdims=True)
    var = jnp.mean((x - mu) ** 2, axis=0, keepdims=True)
    a = gamma / jnp.sqrt(var + 1e-5)
    return jnp.concatenate([a, beta - mu * a], axis=0)


def _dense_body(e_ref, xs_ref, nodes_ref, gamma_ref, beta_ref,
                w1_ref, b1_ref, w2_ref, b2_ref, w3_ref, b3_ref,
                r_ref, s_ref, msg_ref, ac_s):
    @pl.when(pl.program_id(0) == 0)
    def _():
        ac_s[...] = _bn_affine(nodes_ref[...], gamma_ref[...], beta_ref[...])

    x = xs_ref[...] * ac_s[0:1, :] + ac_s[1:2, :]
    h1 = jnp.maximum(e_ref[...] * w1_ref[...] + b1_ref[...], 0.0)
    h2 = jnp.maximum(
        jnp.dot(h1.astype(jnp.bfloat16), w2_ref[...],
                preferred_element_type=jnp.float32) + b2_ref[...], 0.0)
    w = (jnp.dot(h2.astype(jnp.bfloat16), w3_ref[...],
                 preferred_element_type=jnp.float32) + b3_ref[...])
    xr = jnp.dot(x, r_ref[...], preferred_element_type=jnp.float32)
    msg_ref[...] = jnp.dot(xr * w, s_ref[...],
                           preferred_element_type=jnp.float32)


def _dense(e, xs, nodes, gamma2, beta2, w1, b1r, w2b, b2r, w3b, b3r, r, s):
    full = lambda shape: pl.BlockSpec(shape, lambda i: (0, 0))
    return pl.pallas_call(
        _dense_body,
        grid=(E // EB,),
        in_specs=[
            pl.BlockSpec((EB, 1), lambda i: (i, 0)),
            pl.BlockSpec((EB, D), lambda i: (i, 0)),
            full((N, D)), full((1, D)), full((1, D)),
            full((1, H)), full((1, H)),
            full((H, H)), full((1, H)),
            full((H, DD)), full((1, DD)),
            full((D, DD)), full((DD, D)),
        ],
        out_specs=pl.BlockSpec((EB, D), lambda i: (i, 0)),
        out_shape=jax.ShapeDtypeStruct((E, D), jnp.float32),
        scratch_shapes=[pltpu.VMEM((2, D), jnp.float32)],
    )(e, xs, nodes, gamma2, beta2, w1, b1r, w2b, b2r, w3b, b3r, r, s)


def _update_body(aggp_ref, cntp_ref, bias_ref, hin_ref, hout_ref):
    agg = aggp_ref[0, :N, :] + aggp_ref[1, :N, :]
    cnt = cntp_ref[0, :N, 0:1] + cntp_ref[1, :N, 0:1]
    denom = jnp.maximum(cnt, 1.0)
    hout_ref[...] = agg / denom + bias_ref[...] + hin_ref[...]


def _update(aggp, cntp, bias2, hin):
    return pl.pallas_call(
        _update_body,
        out_shape=jax.ShapeDtypeStruct((N, D), jnp.float32),
    )(aggp, cntp, bias2, hin)


def kernel(h, e, edge_index, W1, b1, W2, b2, W3, b3, bias, gamma, beta):
    src2 = edge_index[1].reshape(NW * NCH, CH)
    dst2 = edge_index[0].reshape(NW * NCH, CH)
    b1r = b1.reshape(1, H)
    b2r = b2.reshape(1, H)
    b3r = b3.reshape(1, DD)
    bias2 = bias.reshape(1, D)
    gamma2 = gamma.reshape(1, D)
    beta2 = beta.reshape(1, D)
    w2b = W2.astype(jnp.bfloat16)
    w3b = W3.astype(jnp.bfloat16)
    r = jnp.kron(jnp.eye(D, dtype=jnp.float32),
                 jnp.ones((1, D), jnp.float32))
    s = jnp.kron(jnp.ones((D, 1), jnp.float32),
                 jnp.eye(D, dtype=jnp.float32))

    xs1, cntp = _sc_gather_counts(h, src2, dst2)
    msg1 = _dense(e, xs1.reshape(E, D), h, gamma2, beta2,
                  W1, b1r, w2b, b2r, w3b, b3r, r, s)
    aggp1 = _sc_scatter(dst2, msg1.reshape(NW * NCH, CH, D))
    h2 = _update(aggp1, cntp, bias2, h)
    xs2 = _sc_gather(h2, src2)
    msg2 = _dense(e, xs2.reshape(E, D), h2, gamma2, beta2,
                  W1, b1r, w2b, b2r, w3b, b3r, r, s)
    aggp2 = _sc_scatter(dst2, msg2.reshape(NW * NCH, CH, D))
    return _update(aggp2, cntp, bias2, h2)
